# baseline (speedup 1.0000x reference)
import jax
import jax.numpy as jnp
from jax import lax
from jax.experimental import pallas as pl
from jax.experimental.pallas import tpu as pltpu

_BN_EPS = 1e-5


def _fold_bn(w, conv_b, gamma, beta, mean, var, eps=_BN_EPS):
    inv = gamma / jnp.sqrt(var + eps)
    w_f = w * inv[:, None, None, None]
    b_f = (conv_b - mean) * inv + beta
    return w_f, b_f


def _make_body(*, H, W, L, k):
    r = k // 2

    def _body(x_ref, w1_ref, b1_ref, w2_ref, b2_ref, w3_ref, b3_ref, o_ref):
        f = lax.broadcasted_iota(jnp.int32, (1, L), 1)
        col = f - (f // W) * W
        tail_ok = f < H * W
        z = jnp.bfloat16(0)

        for b in range(x_ref.shape[0]):
            xb = x_ref[b].astype(jnp.bfloat16)

            y1 = jnp.dot(w1_ref[...], xb, preferred_element_type=jnp.float32)
            y1 = jnp.maximum(y1 + b1_ref[...], 0.0)

            y1z = jnp.where(tail_ok, y1, 0.0).astype(jnp.bfloat16)
            c_p1 = jnp.where(col == 0, z, y1z)
            c_m1 = jnp.where(col == W - 1, z, y1z)
            src = {-1: c_m1, 0: y1z, 1: c_p1}

            parts = []
            for i in range(k):
                for j in range(k):
                    di, dj = i - r, j - r
                    delta = di * W + dj
                    if delta == 0:
                        patch = y1z
                    else:
                        patch = pltpu.roll(src[dj], shift=(L - delta) % L,
                                           axis=1)
                    parts.append(patch)
            x2 = jnp.concatenate(parts, axis=0)
            y2 = jnp.dot(w2_ref[...], x2, preferred_element_type=jnp.float32)
            y2 = jnp.maximum(y2 + b2_ref[...], 0.0).astype(jnp.bfloat16)

            y3 = jnp.dot(w3_ref[...], y2, preferred_element_type=jnp.float32)
            o_ref[b] = jnp.maximum(y3 + b3_ref[...], 0.0)

    return _body


def kernel(x, w1, b1, w2, b2, w3, b3, g1, be1, m1, v1,
           g2, be2, m2, v2, g3, be3, m3, v3):
    N, Cin, H, W = x.shape
    Cmid = w1.shape[0]
    Cout = w3.shape[0]
    k = w2.shape[2]
    HW = H * W
    L = ((HW + 127) // 128) * 128
    if L - HW < W + 1:
        L += 128

    w1f, b1f = _fold_bn(w1, b1, g1, be1, m1, v1)
    w2f, b2f = _fold_bn(w2, b2, g2, be2, m2, v2)
    w3f, b3f = _fold_bn(w3, b3, g3, be3, m3, v3)

    w1_2d = w1f.reshape(Cmid, Cin).astype(jnp.bfloat16)
    w2_cat = jnp.transpose(w2f, (0, 2, 3, 1)).reshape(Cmid, k * k * Cmid)
    w2_cat = w2_cat.astype(jnp.bfloat16)
    w3_2d = w3f.reshape(Cout, Cmid).astype(jnp.bfloat16)
    b1_2d = b1f.reshape(Cmid, 1).astype(jnp.float32)
    b2_2d = b2f.reshape(Cmid, 1).astype(jnp.float32)
    b3_2d = b3f.reshape(Cout, 1).astype(jnp.float32)

    x_flat = jnp.pad(x.reshape(N, Cin, HW), ((0, 0), (0, 0), (0, L - HW)))

    B = 1

    out_flat = pl.pallas_call(
        _make_body(H=H, W=W, L=L, k=k),
        out_shape=jax.ShapeDtypeStruct((N, Cout, L), jnp.float32),
        grid=(N // B,),
        in_specs=[
            pl.BlockSpec((B, Cin, L), lambda n: (n, 0, 0)),
            pl.BlockSpec((Cmid, Cin), lambda n: (0, 0)),
            pl.BlockSpec((Cmid, 1), lambda n: (0, 0)),
            pl.BlockSpec((Cmid, k * k * Cmid), lambda n: (0, 0)),
            pl.BlockSpec((Cmid, 1), lambda n: (0, 0)),
            pl.BlockSpec((Cout, Cmid), lambda n: (0, 0)),
            pl.BlockSpec((Cout, 1), lambda n: (0, 0)),
        ],
        out_specs=pl.BlockSpec((B, Cout, L), lambda n: (n, 0, 0)),
        compiler_params=pltpu.CompilerParams(
            dimension_semantics=("parallel",),
            vmem_limit_bytes=64 * 1024 * 1024,
        ),
    )(x_flat, w1_2d, b1_2d, w2_cat, b2_2d, w3_2d, b3_2d)

    return out_flat[:, :, :HW].reshape(N, Cout, H, W)


# speedup vs baseline: 1.1165x; 1.1165x over previous
import jax
import jax.numpy as jnp
from jax import lax
from jax.experimental import pallas as pl
from jax.experimental.pallas import tpu as pltpu

_BN_EPS = 1e-5


def _fold_bn(w, conv_b, gamma, beta, mean, var, eps=_BN_EPS):
    inv = gamma / jnp.sqrt(var + eps)
    w_f = w * inv[:, None, None, None]
    b_f = (conv_b - mean) * inv + beta
    return w_f, b_f


def _make_body(*, H, W, L, k):
    r = k // 2

    def _body(x_ref, w1_ref, b1_ref, w2_ref, b2_ref, w3_ref, b3_ref, o_ref):
        f = lax.broadcasted_iota(jnp.int32, (1, L), 1)
        col = f - (f // W) * W
        tail_ok = f < H * W
        z = jnp.bfloat16(0)

        for b in range(x_ref.shape[0]):
            xb = x_ref[b].astype(jnp.bfloat16)

            y1 = jnp.dot(w1_ref[...], xb, preferred_element_type=jnp.float32)
            y1 = jnp.maximum(y1 + b1_ref[...], 0.0)

            y1z = jnp.where(tail_ok, y1, 0.0).astype(jnp.bfloat16)
            c_p1 = jnp.where(col == 0, z, y1z)
            c_m1 = jnp.where(col == W - 1, z, y1z)
            src = {-1: c_m1, 0: y1z, 1: c_p1}

            parts = []
            for i in range(k):
                for j in range(k):
                    di, dj = i - r, j - r
                    delta = di * W + dj
                    if delta == 0:
                        patch = y1z
                    else:
                        patch = pltpu.roll(src[dj], shift=(L - delta) % L,
                                           axis=1)
                    parts.append(patch)
            x2 = jnp.concatenate(parts, axis=0)
            y2 = jnp.dot(w2_ref[...], x2, preferred_element_type=jnp.float32)
            y2 = jnp.maximum(y2 + b2_ref[...], 0.0).astype(jnp.bfloat16)

            y3 = jnp.dot(w3_ref[...], y2, preferred_element_type=jnp.float32)
            o_ref[b] = jnp.maximum(y3 + b3_ref[...], 0.0)

    return _body


def kernel(x, w1, b1, w2, b2, w3, b3, g1, be1, m1, v1,
           g2, be2, m2, v2, g3, be3, m3, v3):
    N, Cin, H, W = x.shape
    Cmid = w1.shape[0]
    Cout = w3.shape[0]
    k = w2.shape[2]
    HW = H * W
    L = ((HW + 127) // 128) * 128
    if L - HW < W + 1:
        L += 128

    w1f, b1f = _fold_bn(w1, b1, g1, be1, m1, v1)
    w2f, b2f = _fold_bn(w2, b2, g2, be2, m2, v2)
    w3f, b3f = _fold_bn(w3, b3, g3, be3, m3, v3)

    w1_2d = w1f.reshape(Cmid, Cin).astype(jnp.bfloat16)
    w2_cat = jnp.transpose(w2f, (0, 2, 3, 1)).reshape(Cmid, k * k * Cmid)
    w2_cat = w2_cat.astype(jnp.bfloat16)
    w3_2d = w3f.reshape(Cout, Cmid).astype(jnp.bfloat16)
    b1_2d = b1f.reshape(Cmid, 1).astype(jnp.float32)
    b2_2d = b2f.reshape(Cmid, 1).astype(jnp.float32)
    b3_2d = b3f.reshape(Cout, 1).astype(jnp.float32)

    x_flat = x.reshape(N, Cin, HW)

    B = 1

    out_flat = pl.pallas_call(
        _make_body(H=H, W=W, L=L, k=k),
        out_shape=jax.ShapeDtypeStruct((N, Cout, HW), jnp.float32),
        grid=(N // B,),
        in_specs=[
            pl.BlockSpec((B, Cin, L), lambda n: (n, 0, 0)),
            pl.BlockSpec((Cmid, Cin), lambda n: (0, 0)),
            pl.BlockSpec((Cmid, 1), lambda n: (0, 0)),
            pl.BlockSpec((Cmid, k * k * Cmid), lambda n: (0, 0)),
            pl.BlockSpec((Cmid, 1), lambda n: (0, 0)),
            pl.BlockSpec((Cout, Cmid), lambda n: (0, 0)),
            pl.BlockSpec((Cout, 1), lambda n: (0, 0)),
        ],
        out_specs=pl.BlockSpec((B, Cout, L), lambda n: (n, 0, 0)),
        compiler_params=pltpu.CompilerParams(
            dimension_semantics=("parallel",),
            vmem_limit_bytes=64 * 1024 * 1024,
        ),
    )(x_flat, w1_2d, b1_2d, w2_cat, b2_2d, w3_2d, b3_2d)

    return out_flat.reshape(N, Cout, H, W)


# speedup vs baseline: 1.1201x; 1.0032x over previous
import jax
import jax.numpy as jnp
from jax import lax
from jax.experimental import pallas as pl
from jax.experimental.pallas import tpu as pltpu

_BN_EPS = 1e-5


def _fold_bn(w, conv_b, gamma, beta, mean, var, eps=_BN_EPS):
    inv = gamma / jnp.sqrt(var + eps)
    w_f = w * inv[:, None, None, None]
    b_f = (conv_b - mean) * inv + beta
    return w_f, b_f


def _make_body(*, H, W, L, k):
    r = k // 2

    def _body(x_ref, w1_ref, b1_ref, w2_ref, b2_ref, w3_ref, b3_ref, o_ref):
        B = x_ref.shape[0]
        f = lax.broadcasted_iota(jnp.int32, (1, L), 1)
        col1 = f - (f // W) * W
        tail1 = f < H * W
        col = jnp.concatenate([col1] * B, axis=1) if B > 1 else col1
        tail_ok = jnp.concatenate([tail1] * B, axis=1) if B > 1 else tail1
        z = jnp.bfloat16(0)
        LB = B * L

        xb = jnp.concatenate(
            [x_ref[b].astype(jnp.bfloat16) for b in range(B)], axis=1)

        y1 = jnp.dot(w1_ref[...], xb, preferred_element_type=jnp.float32)
        y1 = jnp.maximum(y1 + b1_ref[...], 0.0)

        y1z = jnp.where(tail_ok, y1, 0.0).astype(jnp.bfloat16)
        c_p1 = jnp.where(col == 0, z, y1z)
        c_m1 = jnp.where(col == W - 1, z, y1z)
        src = {-1: c_m1, 0: y1z, 1: c_p1}

        parts = []
        for i in range(k):
            for j in range(k):
                di, dj = i - r, j - r
                delta = di * W + dj
                if delta == 0:
                    patch = y1z
                else:
                    patch = pltpu.roll(src[dj], shift=(LB - delta) % LB,
                                       axis=1)
                parts.append(patch)
        x2 = jnp.concatenate(parts, axis=0)
        y2 = jnp.dot(w2_ref[...], x2, preferred_element_type=jnp.float32)
        y2 = jnp.maximum(y2 + b2_ref[...], 0.0).astype(jnp.bfloat16)

        y3 = jnp.dot(w3_ref[...], y2, preferred_element_type=jnp.float32)
        out = jnp.maximum(y3 + b3_ref[...], 0.0)
        for b in range(B):
            o_ref[b] = out[:, b * L:(b + 1) * L]

    return _body


def kernel(x, w1, b1, w2, b2, w3, b3, g1, be1, m1, v1,
           g2, be2, m2, v2, g3, be3, m3, v3):
    N, Cin, H, W = x.shape
    Cmid = w1.shape[0]
    Cout = w3.shape[0]
    k = w2.shape[2]
    HW = H * W
    L = ((HW + 127) // 128) * 128
    if L - HW < W + 1:
        L += 128

    w1f, b1f = _fold_bn(w1, b1, g1, be1, m1, v1)
    w2f, b2f = _fold_bn(w2, b2, g2, be2, m2, v2)
    w3f, b3f = _fold_bn(w3, b3, g3, be3, m3, v3)

    w1_2d = w1f.reshape(Cmid, Cin).astype(jnp.bfloat16)
    w2_cat = jnp.transpose(w2f, (0, 2, 3, 1)).reshape(Cmid, k * k * Cmid)
    w2_cat = w2_cat.astype(jnp.bfloat16)
    w3_2d = w3f.reshape(Cout, Cmid).astype(jnp.bfloat16)
    b1_2d = b1f.reshape(Cmid, 1).astype(jnp.float32)
    b2_2d = b2f.reshape(Cmid, 1).astype(jnp.float32)
    b3_2d = b3f.reshape(Cout, 1).astype(jnp.float32)

    x_flat = x.reshape(N, Cin, HW)

    B = 2 if N % 2 == 0 else 1

    out_flat = pl.pallas_call(
        _make_body(H=H, W=W, L=L, k=k),
        out_shape=jax.ShapeDtypeStruct((N, Cout, HW), jnp.float32),
        grid=(N // B,),
        in_specs=[
            pl.BlockSpec((B, Cin, L), lambda n: (n, 0, 0)),
            pl.BlockSpec((Cmid, Cin), lambda n: (0, 0)),
            pl.BlockSpec((Cmid, 1), lambda n: (0, 0)),
            pl.BlockSpec((Cmid, k * k * Cmid), lambda n: (0, 0)),
            pl.BlockSpec((Cmid, 1), lambda n: (0, 0)),
            pl.BlockSpec((Cout, Cmid), lambda n: (0, 0)),
            pl.BlockSpec((Cout, 1), lambda n: (0, 0)),
        ],
        out_specs=pl.BlockSpec((B, Cout, L), lambda n: (n, 0, 0)),
        compiler_params=pltpu.CompilerParams(
            dimension_semantics=("parallel",),
            vmem_limit_bytes=64 * 1024 * 1024,
        ),
    )(x_flat, w1_2d, b1_2d, w2_cat, b2_2d, w3_2d, b3_2d)

    return out_flat.reshape(N, Cout, H, W)
